# baseline (speedup 1.0000x reference)
import functools

import jax
import jax.numpy as jnp
from jax import lax
from jax.experimental import pallas as pl

B = 1024
T = 80
H = 256
DICT = 80
G = 4 * H


def _lstm_body(sent_ref, emb_ref, w_ih0t_ref, w_hh0t_ref, bias0_ref,
               w1cat_ref, bias1_ref, w_dect_ref, b_dec_ref,
               out_ref):
    f32 = jnp.float32
    bf16 = jnp.bfloat16
    table0 = (jnp.dot(emb_ref[...], w_ih0t_ref[...],
                      preferred_element_type=f32)
              + bias0_ref[...]).astype(bf16)

    vocab_iota = lax.broadcasted_iota(jnp.int32, (DICT, B), 0)

    def cell(g, c):
        si = 0.5 * jnp.tanh(g[:, 0 * H:1 * H]) + 0.5
        sf = 0.5 * jnp.tanh(g[:, 1 * H:2 * H]) + 0.5
        tg = jnp.tanh(g[:, 2 * H:3 * H])
        so = 0.5 * jnp.tanh(g[:, 3 * H:4 * H]) + 0.5
        c = sf * c + si * tg
        h = (so * jnp.tanh(c)).astype(bf16)
        return h, c

    def onehot_dot(t):
        row = sent_ref[pl.ds(t, 1), :]
        onehot_t = (row == vocab_iota).astype(bf16)
        return lax.dot_general(onehot_t, table0,
                               (((0,), (0,)), ((), ())),
                               preferred_element_type=f32)

    def step(t, carry):
        h0, c0, h1, c1 = carry
        g0 = onehot_dot(t) + jnp.dot(h0, w_hh0t_ref[...],
                                     preferred_element_type=f32)
        h0, c0 = cell(g0, c0)
        hcat = jnp.concatenate([h0, h1], axis=1)
        g1 = (jnp.dot(hcat, w1cat_ref[...], preferred_element_type=f32)
              + bias1_ref[...])
        h1, c1 = cell(g1, c1)
        return h0, c0, h1, c1

    UNROLL = 16

    def stepn(i, carry):
        for k in range(UNROLL):
            carry = step(UNROLL * i + k, carry)
        return carry

    zf, zb = jnp.zeros((B, H), f32), jnp.zeros((B, H), bf16)
    h0, c0, h1, c1 = lax.fori_loop(0, T // UNROLL, stepn, (zb, zf, zb, zf))
    out_ref[...] = (jnp.dot(h1, w_dect_ref[...], preferred_element_type=f32)
                    + b_dec_ref[...])


@functools.partial(jax.jit, static_argnums=())
def kernel(sentence, emb, w_ih0, w_hh0, b_ih0, b_hh0,
           w_ih1, w_hh1, b_ih1, b_hh1, W_dec, b_dec):
    sent_t = jnp.transpose(sentence.astype(jnp.int32), (1, 0))
    gscale = jnp.concatenate(
        [jnp.full((1, H), 0.5, jnp.float32),
         jnp.full((1, H), 0.5, jnp.float32),
         jnp.ones((1, H), jnp.float32),
         jnp.full((1, H), 0.5, jnp.float32)], axis=1)
    bias0 = (b_ih0 + b_hh0).reshape(1, G) * gscale
    bias1 = (b_ih1 + b_hh1).reshape(1, G) * gscale
    _call = pl.pallas_call(
        _lstm_body,
        out_shape=jax.ShapeDtypeStruct((B, DICT), jnp.float32),
    )
    w1cat = jnp.concatenate([w_ih1.T * gscale, w_hh1.T * gscale],
                            axis=0).astype(jnp.bfloat16)
    return_args = (sent_t, emb, w_ih0.T * gscale,
                   (w_hh0.T * gscale).astype(jnp.bfloat16), bias0,
                   w1cat, bias1,
                   W_dec.T.astype(jnp.bfloat16), b_dec.reshape(1, DICT))
    return _call(*return_args)


# speedup vs baseline: 1.1027x; 1.1027x over previous
import functools

import jax
import jax.numpy as jnp
from jax import lax
from jax.experimental import pallas as pl

B = 1024
T = 80
H = 256
DICT = 80
G = 4 * H


def _lstm_body(sent_ref, emb_ref, w_ih0t_ref, w_hh0t_ref, bias0_ref,
               w1cat_ref, bias1_ref, w_dect_ref, b_dec_ref,
               out_ref):
    f32 = jnp.float32
    bf16 = jnp.bfloat16
    table0 = (jnp.dot(emb_ref[...], w_ih0t_ref[...],
                      preferred_element_type=f32)
              + bias0_ref[...]).astype(bf16)

    vocab_iota = lax.broadcasted_iota(jnp.int32, (DICT, B), 0)

    def cell(g, c):
        si = 0.5 * jnp.tanh(g[:, 0 * H:1 * H]) + 0.5
        sf = 0.5 * jnp.tanh(g[:, 1 * H:2 * H]) + 0.5
        tg = jnp.tanh(g[:, 2 * H:3 * H])
        so = 0.5 * jnp.tanh(g[:, 3 * H:4 * H]) + 0.5
        c = sf * c + si * tg
        h = (so * jnp.tanh(c)).astype(bf16)
        return h, c

    def onehot_dot(t):
        row = sent_ref[pl.ds(t, 1), :]
        onehot_t = (row == vocab_iota).astype(bf16)
        return lax.dot_general(onehot_t, table0,
                               (((0,), (0,)), ((), ())),
                               preferred_element_type=f32)

    def step(t, carry):
        h0, c0, h1, c1 = carry
        g0 = onehot_dot(t) + jnp.dot(h0, w_hh0t_ref[...],
                                     preferred_element_type=f32)
        h0, c0 = cell(g0, c0)
        hcat = jnp.concatenate([h0, h1], axis=1)
        g1 = (jnp.dot(hcat, w1cat_ref[...], preferred_element_type=f32)
              + bias1_ref[...])
        h1, c1 = cell(g1, c1)
        return h0, c0, h1, c1

    UNROLL = 10

    def stepn(i, carry):
        for k in range(UNROLL):
            carry = step(UNROLL * i + k, carry)
        return carry

    zf, zb = jnp.zeros((B, H), f32), jnp.zeros((B, H), bf16)
    h0, c0, h1, c1 = lax.fori_loop(0, T // UNROLL, stepn, (zb, zf, zb, zf))
    out_ref[...] = (jnp.dot(h1, w_dect_ref[...], preferred_element_type=f32)
                    + b_dec_ref[...])


@functools.partial(jax.jit, static_argnums=())
def kernel(sentence, emb, w_ih0, w_hh0, b_ih0, b_hh0,
           w_ih1, w_hh1, b_ih1, b_hh1, W_dec, b_dec):
    sent_t = jnp.transpose(sentence.astype(jnp.int32), (1, 0))
    gscale = jnp.concatenate(
        [jnp.full((1, H), 0.5, jnp.float32),
         jnp.full((1, H), 0.5, jnp.float32),
         jnp.ones((1, H), jnp.float32),
         jnp.full((1, H), 0.5, jnp.float32)], axis=1)
    bias0 = (b_ih0 + b_hh0).reshape(1, G) * gscale
    bias1 = (b_ih1 + b_hh1).reshape(1, G) * gscale
    _call = pl.pallas_call(
        _lstm_body,
        out_shape=jax.ShapeDtypeStruct((B, DICT), jnp.float32),
    )
    w1cat = jnp.concatenate([w_ih1.T * gscale, w_hh1.T * gscale],
                            axis=0).astype(jnp.bfloat16)
    return_args = (sent_t, emb, w_ih0.T * gscale,
                   (w_hh0.T * gscale).astype(jnp.bfloat16), bias0,
                   w1cat, bias1,
                   W_dec.T.astype(jnp.bfloat16), b_dec.reshape(1, DICT))
    return _call(*return_args)


# speedup vs baseline: 1.1251x; 1.0202x over previous
import functools

import jax
import jax.numpy as jnp
from jax import lax
from jax.experimental import pallas as pl

B = 1024
T = 80
H = 256
DICT = 80
G = 4 * H


def _lstm_body(sent_ref, emb_ref, w_ih0t_ref, w_hh0t_ref, bias0_ref,
               w1cat_ref, bias1_ref, w_dect_ref, b_dec_ref,
               out_ref):
    f32 = jnp.float32
    bf16 = jnp.bfloat16
    table0 = (jnp.dot(emb_ref[...], w_ih0t_ref[...],
                      preferred_element_type=f32)
              + bias0_ref[...]).astype(bf16)

    vocab_iota = lax.broadcasted_iota(jnp.int32, (DICT, B), 0)

    def cell(g, c):
        ti = jnp.tanh(g[:, 0 * H:1 * H])
        tf = jnp.tanh(g[:, 1 * H:2 * H])
        tg = jnp.tanh(g[:, 2 * H:3 * H])
        to = jnp.tanh(g[:, 3 * H:4 * H])
        c = 0.5 * ((tf + 1.0) * c + (ti + 1.0) * tg)
        h2 = ((to + 1.0) * jnp.tanh(c)).astype(bf16)
        return h2, c

    def onehot_dot(t):
        row = sent_ref[pl.ds(t, 1), :]
        onehot_t = (row == vocab_iota).astype(bf16)
        return lax.dot_general(onehot_t, table0,
                               (((0,), (0,)), ((), ())),
                               preferred_element_type=f32)

    def step(t, carry):
        h0, c0, h1, c1 = carry
        g0 = onehot_dot(t) + jnp.dot(h0, w_hh0t_ref[...],
                                     preferred_element_type=f32)
        h0, c0 = cell(g0, c0)
        hcat = jnp.concatenate([h0, h1], axis=1)
        g1 = (jnp.dot(hcat, w1cat_ref[...], preferred_element_type=f32)
              + bias1_ref[...])
        h1, c1 = cell(g1, c1)
        return h0, c0, h1, c1

    UNROLL = 10

    def stepn(i, carry):
        for k in range(UNROLL):
            carry = step(UNROLL * i + k, carry)
        return carry

    zf, zb = jnp.zeros((B, H), f32), jnp.zeros((B, H), bf16)
    h0, c0, h1, c1 = lax.fori_loop(0, T // UNROLL, stepn, (zb, zf, zb, zf))
    out_ref[...] = (jnp.dot(h1, w_dect_ref[...], preferred_element_type=f32)
                    + b_dec_ref[...])


@functools.partial(jax.jit, static_argnums=())
def kernel(sentence, emb, w_ih0, w_hh0, b_ih0, b_hh0,
           w_ih1, w_hh1, b_ih1, b_hh1, W_dec, b_dec):
    sent_t = jnp.transpose(sentence.astype(jnp.int32), (1, 0))
    gscale = jnp.concatenate(
        [jnp.full((1, H), 0.5, jnp.float32),
         jnp.full((1, H), 0.5, jnp.float32),
         jnp.ones((1, H), jnp.float32),
         jnp.full((1, H), 0.5, jnp.float32)], axis=1)
    bias0 = (b_ih0 + b_hh0).reshape(1, G) * gscale
    bias1 = (b_ih1 + b_hh1).reshape(1, G) * gscale
    _call = pl.pallas_call(
        _lstm_body,
        out_shape=jax.ShapeDtypeStruct((B, DICT), jnp.float32),
    )
    w1cat = jnp.concatenate([w_ih1.T * gscale, w_hh1.T * gscale],
                            axis=0).astype(jnp.bfloat16) * 0.5
    return_args = (sent_t, emb, w_ih0.T * gscale,
                   (w_hh0.T * gscale).astype(jnp.bfloat16) * 0.5, bias0,
                   w1cat, bias1,
                   W_dec.T.astype(jnp.bfloat16) * 0.5, b_dec.reshape(1, DICT))
    return _call(*return_args)
